# baseline (speedup 1.0000x reference)
import functools

import jax
import jax.numpy as jnp
from jax import lax
from jax.experimental import pallas as pl
from jax.experimental.pallas import tpu as pltpu
from jax.experimental.pallas import tpu_sc as plsc

N_NODES = 10000
D_IN = 128
HALF = 5000
N_ACC = 10016
N_ACC_H = HALF
NUM_TILES = 32
CHUNK = 128
NCH_DEG = 80
EPT = NCH_DEG * CHUNK
CAP = EPT + CHUNK
NB = 5
BR = 1000


def _sc_mesh():
    return plsc.VectorSubcoreMesh(core_axis_name="c", subcore_axis_name="s")


def _part_kernel(src_hbm, dst_hbm, zeros_hbm,
                 deg_out, psrc_out, pdst_out, cnt_out,
                 srcv, dstv, lo_s, lo_d, hi_s, hi_d, ones_v, cntv, deg_sh):
    cid = lax.axis_index("c")
    sid = lax.axis_index("s")
    wid = cid * 16 + sid

    @pl.when(sid == 0)
    def _():
        pltpu.sync_copy(zeros_hbm, deg_sh)

    for j in range(CHUNK // 16):
        ones_v[pl.ds(j * 16, 16)] = jnp.ones((16,), jnp.float32)
    pltpu.sync_copy(src_hbm.at[wid], srcv)
    pltpu.sync_copy(dst_hbm.at[wid], dstv)
    plsc.subcore_barrier()

    def body(c, carry):
        pltpu.sync_copy(ones_v, deg_sh.at[dstv.at[c]], add=True)
        return carry

    lax.fori_loop(0, NCH_DEG, body, 0)

    def pbody(i, carry):
        lo_cnt, hi_cnt = carry
        c = i // 8
        g = (i % 8) * 16
        vd = dstv[c, pl.ds(g, 16)]
        vs = srcv[c, pl.ds(g, 16)]
        mall = jnp.ones((16,), jnp.bool_)
        vdau, vsa, _ = plsc.sort_key_val(
            plsc.bitcast(vd, jnp.uint32), vs, mask=mall)
        vda = plsc.bitcast(vdau, jnp.int32)
        vddu, vsd, _ = plsc.sort_key_val(
            plsc.bitcast(vd, jnp.uint32), vs, mask=mall, descending=True)
        vdd = plsc.bitcast(vddu, jnp.int32)
        lo_d[pl.ds(lo_cnt, 16)] = vda
        lo_s[pl.ds(lo_cnt, 16)] = vsa
        vr = vdd - HALF
        hi_d[pl.ds(hi_cnt, 16)] = jnp.where(vr >= HALF, vr - HALF, vr)
        hi_s[pl.ds(hi_cnt, 16)] = vsd
        nlo = plsc.all_reduce_population_count(vd < HALF)[0]
        return lo_cnt + nlo, hi_cnt + (16 - nlo)

    zero = jnp.int32(0)
    lo_cnt, hi_cnt = lax.fori_loop(0, NCH_DEG * 8, pbody, (zero, zero))

    junk_d = jnp.arange(16, dtype=jnp.int32)
    junk_s = N_NODES + (jnp.arange(16, dtype=jnp.int32) % 8)

    def jbody(k, carry):
        lo_d[pl.ds(lo_cnt + k * 16, 16)] = junk_d
        lo_s[pl.ds(lo_cnt + k * 16, 16)] = junk_s
        hi_d[pl.ds(hi_cnt + k * 16, 16)] = junk_d
        hi_s[pl.ds(hi_cnt + k * 16, 16)] = junk_s
        return carry

    lax.fori_loop(0, CHUNK // 16, jbody, 0)

    cntv[0, pl.ds(0, 16)] = jnp.full((16,), lo_cnt, jnp.int32)
    cntv[1, pl.ds(0, 16)] = jnp.full((16,), hi_cnt, jnp.int32)
    pltpu.sync_copy(lo_s, psrc_out.at[0, wid])
    pltpu.sync_copy(lo_d, pdst_out.at[0, wid])
    pltpu.sync_copy(hi_s, psrc_out.at[1, wid])
    pltpu.sync_copy(hi_d, pdst_out.at[1, wid])
    pltpu.sync_copy(cntv, cnt_out.at[wid])
    plsc.subcore_barrier()

    @pl.when(sid == 0)
    def _():
        pltpu.sync_copy(deg_sh, deg_out.at[cid])


def _scatter_kernel(z_hbm, psrc_hbm, pdst_hbm, cnt_hbm, zeros_hbm, acc_out,
                    idxs, idxd, bufs, acc_sh, ise, gsem, ssem):
    cid = lax.axis_index("c")
    sid = lax.axis_index("s")
    p0 = 2 * sid
    p1 = 2 * sid + 1

    @pl.when(sid == 0)
    def _():
        pltpu.sync_copy(zeros_hbm, acc_sh)

    pltpu.sync_copy(cnt_hbm.at[pl.ds(sid * 64, 64)], idxs.at[0, pl.ds(0, 64)])
    plsc.subcore_barrier()

    c0 = idxs[0, pl.ds(cid * 16, 16)][0]
    c1 = idxs[0, pl.ds(32 + cid * 16, 16)][0]
    t0 = (c0 + CHUNK - 1) // CHUNK
    t1 = (c1 + CHUNK - 1) // CHUNK
    total = t0 + t1

    def step(s, carry):
        j = lax.rem(s, NB)

        @pl.when(jnp.logical_and(s >= NB, s - NB < total))
        def _():
            pltpu.make_async_copy(
                bufs.at[j], acc_sh.at[idxd.at[j]], ssem.at[j]).wait()

        @pl.when(s < total)
        def _():
            in0 = s < t0
            pt = jnp.where(in0, p0, p1)
            off = jnp.where(in0, s, s - t0) * CHUNK
            pltpu.async_copy(
                psrc_hbm.at[cid, pt, pl.ds(off, CHUNK)], idxs.at[j],
                ise.at[j])
            pltpu.async_copy(
                pdst_hbm.at[cid, pt, pl.ds(off, CHUNK)], idxd.at[j],
                ise.at[j])

        @pl.when(jnp.logical_and(s >= 2, s - 2 < total))
        def _():
            jg = lax.rem(s - 2, NB)
            pltpu.make_async_copy(
                psrc_hbm.at[cid, p0, pl.ds(0, CHUNK)], idxs.at[jg],
                ise.at[jg]).wait()
            pltpu.make_async_copy(
                pdst_hbm.at[cid, p0, pl.ds(0, CHUNK)], idxd.at[jg],
                ise.at[jg]).wait()
            pltpu.async_copy(
                z_hbm.at[idxs.at[jg]], bufs.at[jg], gsem.at[jg])

        @pl.when(jnp.logical_and(s >= 4, s - 4 < total))
        def _():
            js = lax.rem(s - 4, NB)
            pltpu.make_async_copy(
                z_hbm.at[idxs.at[js]], bufs.at[js], gsem.at[js]).wait()
            pltpu.async_copy(
                bufs.at[js], acc_sh.at[idxd.at[js]], ssem.at[js],
                add=True)

        return carry

    lax.fori_loop(0, total + NB, step, 0)
    plsc.subcore_barrier()

    @pl.when(sid == 0)
    def _():
        pltpu.sync_copy(acc_sh, acc_out.at[cid])


def _scale_kernel(x_ref, d_ref, z_ref):
    deg = d_ref[:, 0:1] + d_ref[:, 1:2] + 1.0
    z_ref[...] = x_ref[...] * lax.rsqrt(deg)


def _out_kernel(a_ref, z_ref, d_ref, w_ref, b_ref, al_ref, o_ref):
    deg = d_ref[:, 0:1] + d_ref[:, 1:2] + 1.0
    s = (a_ref[0] + z_ref[...]) * lax.rsqrt(deg)
    o = jnp.dot(s, w_ref[...], preferred_element_type=jnp.float32)
    o = o + b_ref[...]
    o_ref[...] = jnp.where(o > 0, o, al_ref[...] * o)


def kernel(x, edge_index, W, b, alpha):
    n, d_in = x.shape
    e = edge_index.shape[1]
    d_out = W.shape[1]

    e_pad = NUM_TILES * EPT
    pad16 = jnp.arange(e_pad - e, dtype=jnp.int32) % 16
    src_p = jnp.concatenate([edge_index[0], n + pad16 % 8]).reshape(
        NUM_TILES, NCH_DEG, CHUNK)
    dst_p = jnp.concatenate([edge_index[1], n + pad16]).reshape(
        NUM_TILES, NCH_DEG, CHUNK)

    zeros_deg = jnp.zeros((N_ACC,), jnp.float32)
    zeros_acc = jnp.zeros((N_ACC_H, D_IN), jnp.float32)

    part_fn = pl.kernel(
        _part_kernel,
        out_type=(
            jax.ShapeDtypeStruct((2, N_ACC), jnp.float32),
            jax.ShapeDtypeStruct((2, NUM_TILES, CAP), jnp.int32),
            jax.ShapeDtypeStruct((2, NUM_TILES, CAP), jnp.int32),
            jax.ShapeDtypeStruct((NUM_TILES, 2, 16), jnp.int32),
        ),
        mesh=_sc_mesh(),
        compiler_params=pltpu.CompilerParams(needs_layout_passes=False),
        scratch_types=[
            pltpu.VMEM((NCH_DEG, CHUNK), jnp.int32),
            pltpu.VMEM((NCH_DEG, CHUNK), jnp.int32),
            pltpu.VMEM((CAP,), jnp.int32),
            pltpu.VMEM((CAP,), jnp.int32),
            pltpu.VMEM((CAP,), jnp.int32),
            pltpu.VMEM((CAP,), jnp.int32),
            pltpu.VMEM((CHUNK,), jnp.float32),
            pltpu.VMEM((2, 16), jnp.int32),
            pltpu.VMEM_SHARED((N_ACC,), jnp.float32),
        ],
    )
    deg2, psrc, pdst, cnt = part_fn(src_p, dst_p, zeros_deg)

    deg_pair = deg2[:, :n].T

    grid = n // BR
    z = pl.pallas_call(
        _scale_kernel,
        grid=(grid,),
        in_specs=[
            pl.BlockSpec((BR, d_in), lambda i: (i, 0)),
            pl.BlockSpec((BR, 2), lambda i: (i, 0)),
        ],
        out_specs=pl.BlockSpec((BR, d_in), lambda i: (i, 0)),
        out_shape=jax.ShapeDtypeStruct((n, d_in), jnp.float32),
    )(x, deg_pair)

    scatter_fn = pl.kernel(
        _scatter_kernel,
        out_type=jax.ShapeDtypeStruct((2, N_ACC_H, D_IN), jnp.float32),
        mesh=_sc_mesh(),
        scratch_types=[
            pltpu.VMEM((NB, CHUNK), jnp.int32),
            pltpu.VMEM((NB, CHUNK), jnp.int32),
            pltpu.VMEM((NB, CHUNK, D_IN), jnp.float32),
            pltpu.VMEM_SHARED((N_ACC_H, D_IN), jnp.float32),
            pltpu.SemaphoreType.DMA((NB,)),
            pltpu.SemaphoreType.DMA((NB,)),
            pltpu.SemaphoreType.DMA((NB,)),
        ],
    )
    z_p = jnp.pad(z, ((0, 8), (0, 0)))
    acc = scatter_fn(z_p, psrc, pdst, cnt.reshape(-1), zeros_acc)

    out = pl.pallas_call(
        _out_kernel,
        grid=(grid,),
        in_specs=[
            pl.BlockSpec((1, BR, d_in), lambda i: (i // 5, i % 5, 0)),
            pl.BlockSpec((BR, d_in), lambda i: (i, 0)),
            pl.BlockSpec((BR, 2), lambda i: (i, 0)),
            pl.BlockSpec((d_in, d_out), lambda i: (0, 0)),
            pl.BlockSpec((1, d_out), lambda i: (0, 0)),
            pl.BlockSpec((1, d_out), lambda i: (0, 0)),
        ],
        out_specs=pl.BlockSpec((BR, d_out), lambda i: (i, 0)),
        out_shape=jax.ShapeDtypeStruct((n, d_out), jnp.float32),
    )(acc, z, deg_pair, W, b.reshape(1, d_out), alpha.reshape(1, d_out))
    return out


# speedup vs baseline: 1.0160x; 1.0160x over previous
import functools

import jax
import jax.numpy as jnp
from jax import lax
from jax.experimental import pallas as pl
from jax.experimental.pallas import tpu as pltpu
from jax.experimental.pallas import tpu_sc as plsc

N_NODES = 10000
D_IN = 128
HALF = 5000
N_ACC = 10016
N_ACC_H = HALF
NUM_TILES = 32
CHUNK = 128
NCH_DEG = 80
EPT = NCH_DEG * CHUNK
CAP = EPT + CHUNK
NB = 4
NI = 8
BR = 1000


def _sc_mesh():
    return plsc.VectorSubcoreMesh(core_axis_name="c", subcore_axis_name="s")


def _part_kernel(src_hbm, dst_hbm, zeros_hbm,
                 deg_out, psrc_out, pdst_out, cnt_out,
                 srcv, dstv, lo_s, lo_d, hi_s, hi_d, ones_v, cntv, deg_sh,
                 hsem):
    cid = lax.axis_index("c")
    sid = lax.axis_index("s")
    wid = cid * 16 + sid

    @pl.when(sid == 0)
    def _():
        pltpu.sync_copy(zeros_hbm, deg_sh)

    for j in range(CHUNK // 16):
        ones_v[pl.ds(j * 16, 16)] = jnp.ones((16,), jnp.float32)
    pltpu.sync_copy(src_hbm.at[wid], srcv)
    pltpu.sync_copy(dst_hbm.at[wid], dstv)
    plsc.subcore_barrier()

    def body(c, carry):
        pltpu.async_copy(ones_v, deg_sh.at[dstv.at[c]], hsem, add=True)
        return carry

    lax.fori_loop(0, NCH_DEG, body, 0)

    def pbody(i, carry):
        lo_cnt, hi_cnt = carry
        c = i // 8
        g = (i % 8) * 16
        vd = dstv[c, pl.ds(g, 16)]
        vs = srcv[c, pl.ds(g, 16)]
        mall = jnp.ones((16,), jnp.bool_)
        vdau, vsa, _ = plsc.sort_key_val(
            plsc.bitcast(vd, jnp.uint32), vs, mask=mall)
        vda = plsc.bitcast(vdau, jnp.int32)
        vdd = lax.rev(vda, (0,))
        vsd = lax.rev(vsa, (0,))
        lo_d[pl.ds(lo_cnt, 16)] = vda
        lo_s[pl.ds(lo_cnt, 16)] = vsa
        vr = vdd - HALF
        hi_d[pl.ds(hi_cnt, 16)] = jnp.where(vr >= HALF, vr - HALF, vr)
        hi_s[pl.ds(hi_cnt, 16)] = vsd
        nlo = plsc.all_reduce_population_count(vd < HALF)[0]
        return lo_cnt + nlo, hi_cnt + (16 - nlo)

    zero = jnp.int32(0)
    lo_cnt, hi_cnt = lax.fori_loop(0, NCH_DEG * 8, pbody, (zero, zero))

    def hdrain(c, carry):
        pltpu.make_async_copy(ones_v, deg_sh.at[dstv.at[c]], hsem).wait()
        return carry

    lax.fori_loop(0, NCH_DEG, hdrain, 0)

    junk_d = jnp.arange(16, dtype=jnp.int32)
    junk_s = N_NODES + (jnp.arange(16, dtype=jnp.int32) % 8)

    def jbody(k, carry):
        lo_d[pl.ds(lo_cnt + k * 16, 16)] = junk_d
        lo_s[pl.ds(lo_cnt + k * 16, 16)] = junk_s
        hi_d[pl.ds(hi_cnt + k * 16, 16)] = junk_d
        hi_s[pl.ds(hi_cnt + k * 16, 16)] = junk_s
        return carry

    lax.fori_loop(0, CHUNK // 16, jbody, 0)

    cntv[0, pl.ds(0, 16)] = jnp.full((16,), lo_cnt, jnp.int32)
    cntv[1, pl.ds(0, 16)] = jnp.full((16,), hi_cnt, jnp.int32)
    pltpu.sync_copy(lo_s, psrc_out.at[0, wid])
    pltpu.sync_copy(lo_d, pdst_out.at[0, wid])
    pltpu.sync_copy(hi_s, psrc_out.at[1, wid])
    pltpu.sync_copy(hi_d, pdst_out.at[1, wid])
    pltpu.sync_copy(cntv, cnt_out.at[wid])
    plsc.subcore_barrier()

    @pl.when(sid == 0)
    def _():
        pltpu.sync_copy(deg_sh, deg_out.at[cid])


def _scatter_kernel(z_hbm, psrc_hbm, pdst_hbm, cnt_hbm, zeros_hbm, acc_out,
                    idxs, idxd, bufs, acc_sh, ise, gsem, ssem):
    cid = lax.axis_index("c")
    sid = lax.axis_index("s")
    p0 = 2 * sid
    p1 = 2 * sid + 1

    @pl.when(sid == 0)
    def _():
        pltpu.sync_copy(zeros_hbm, acc_sh)

    pltpu.sync_copy(cnt_hbm.at[pl.ds(sid * 64, 64)], idxs.at[0, pl.ds(0, 64)])
    plsc.subcore_barrier()

    c0 = idxs[0, pl.ds(cid * 16, 16)][0]
    c1 = idxs[0, pl.ds(32 + cid * 16, 16)][0]
    t0 = (c0 + CHUNK - 1) // CHUNK
    t1 = (c1 + CHUNK - 1) // CHUNK
    total = t0 + t1

    def step(s, carry):
        @pl.when(jnp.logical_and(s >= 8, s - 8 < total))
        def _():
            jd = lax.rem(s - 8, NB)
            pltpu.make_async_copy(
                bufs.at[jd], acc_sh.at[idxd.at[lax.rem(s - 8, NI)]],
                ssem.at[jd]).wait()

        @pl.when(s < total)
        def _():
            ji = lax.rem(s, NI)
            in0 = s < t0
            pt = jnp.where(in0, p0, p1)
            off = jnp.where(in0, s, s - t0) * CHUNK
            pltpu.async_copy(
                psrc_hbm.at[cid, pt, pl.ds(off, CHUNK)], idxs.at[ji],
                ise.at[ji])
            pltpu.async_copy(
                pdst_hbm.at[cid, pt, pl.ds(off, CHUNK)], idxd.at[ji],
                ise.at[ji])

        @pl.when(jnp.logical_and(s >= 4, s - 4 < total))
        def _():
            jg = lax.rem(s - 4, NB)
            jgi = lax.rem(s - 4, NI)
            pltpu.make_async_copy(
                psrc_hbm.at[cid, p0, pl.ds(0, CHUNK)], idxs.at[jgi],
                ise.at[jgi]).wait()
            pltpu.make_async_copy(
                pdst_hbm.at[cid, p0, pl.ds(0, CHUNK)], idxd.at[jgi],
                ise.at[jgi]).wait()
            pltpu.async_copy(
                z_hbm.at[idxs.at[jgi]], bufs.at[jg], gsem.at[jg])

        @pl.when(jnp.logical_and(s >= 6, s - 6 < total))
        def _():
            js = lax.rem(s - 6, NB)
            jsi = lax.rem(s - 6, NI)
            pltpu.make_async_copy(
                z_hbm.at[idxs.at[jsi]], bufs.at[js], gsem.at[js]).wait()
            pltpu.async_copy(
                bufs.at[js], acc_sh.at[idxd.at[jsi]], ssem.at[js],
                add=True)

        return carry

    lax.fori_loop(0, total + 8, step, 0)
    plsc.subcore_barrier()

    @pl.when(sid == 0)
    def _():
        pltpu.sync_copy(acc_sh, acc_out.at[cid])


def _scale_kernel(x_ref, d_ref, z_ref):
    deg = d_ref[:, 0:1] + d_ref[:, 1:2] + 1.0
    z_ref[...] = x_ref[...] * lax.rsqrt(deg)


def _out_kernel(a_ref, z_ref, d_ref, w_ref, b_ref, al_ref, o_ref):
    deg = d_ref[:, 0:1] + d_ref[:, 1:2] + 1.0
    s = (a_ref[0] + z_ref[...]) * lax.rsqrt(deg)
    o = jnp.dot(s, w_ref[...], preferred_element_type=jnp.float32)
    o = o + b_ref[...]
    o_ref[...] = jnp.where(o > 0, o, al_ref[...] * o)


def kernel(x, edge_index, W, b, alpha):
    n, d_in = x.shape
    e = edge_index.shape[1]
    d_out = W.shape[1]

    e_pad = NUM_TILES * EPT
    pad16 = jnp.arange(e_pad - e, dtype=jnp.int32) % 16
    src_p = jnp.concatenate([edge_index[0], n + pad16 % 8]).reshape(
        NUM_TILES, NCH_DEG, CHUNK)
    dst_p = jnp.concatenate([edge_index[1], n + pad16]).reshape(
        NUM_TILES, NCH_DEG, CHUNK)

    zeros_deg = jnp.zeros((N_ACC,), jnp.float32)
    zeros_acc = jnp.zeros((N_ACC_H, D_IN), jnp.float32)

    part_fn = pl.kernel(
        _part_kernel,
        out_type=(
            jax.ShapeDtypeStruct((2, N_ACC), jnp.float32),
            jax.ShapeDtypeStruct((2, NUM_TILES, CAP), jnp.int32),
            jax.ShapeDtypeStruct((2, NUM_TILES, CAP), jnp.int32),
            jax.ShapeDtypeStruct((NUM_TILES, 2, 16), jnp.int32),
        ),
        mesh=_sc_mesh(),
        compiler_params=pltpu.CompilerParams(needs_layout_passes=False),
        scratch_types=[
            pltpu.VMEM((NCH_DEG, CHUNK), jnp.int32),
            pltpu.VMEM((NCH_DEG, CHUNK), jnp.int32),
            pltpu.VMEM((CAP,), jnp.int32),
            pltpu.VMEM((CAP,), jnp.int32),
            pltpu.VMEM((CAP,), jnp.int32),
            pltpu.VMEM((CAP,), jnp.int32),
            pltpu.VMEM((CHUNK,), jnp.float32),
            pltpu.VMEM((2, 16), jnp.int32),
            pltpu.VMEM_SHARED((N_ACC,), jnp.float32),
            pltpu.SemaphoreType.DMA,
        ],
    )
    deg2, psrc, pdst, cnt = part_fn(src_p, dst_p, zeros_deg)

    deg_pair = deg2[:, :n].T

    grid = n // BR
    z = pl.pallas_call(
        _scale_kernel,
        grid=(grid,),
        in_specs=[
            pl.BlockSpec((BR, d_in), lambda i: (i, 0)),
            pl.BlockSpec((BR, 2), lambda i: (i, 0)),
        ],
        out_specs=pl.BlockSpec((BR, d_in), lambda i: (i, 0)),
        out_shape=jax.ShapeDtypeStruct((n, d_in), jnp.float32),
    )(x, deg_pair)

    scatter_fn = pl.kernel(
        _scatter_kernel,
        out_type=jax.ShapeDtypeStruct((2, N_ACC_H, D_IN), jnp.float32),
        mesh=_sc_mesh(),
        scratch_types=[
            pltpu.VMEM((NI, CHUNK), jnp.int32),
            pltpu.VMEM((NI, CHUNK), jnp.int32),
            pltpu.VMEM((NB, CHUNK, D_IN), jnp.float32),
            pltpu.VMEM_SHARED((N_ACC_H, D_IN), jnp.float32),
            pltpu.SemaphoreType.DMA((NI,)),
            pltpu.SemaphoreType.DMA((NB,)),
            pltpu.SemaphoreType.DMA((NB,)),
        ],
    )
    z_p = jnp.pad(z, ((0, 8), (0, 0)))
    acc = scatter_fn(z_p, psrc, pdst, cnt.reshape(-1), zeros_acc)

    out = pl.pallas_call(
        _out_kernel,
        grid=(grid,),
        in_specs=[
            pl.BlockSpec((1, BR, d_in), lambda i: (i // 5, i % 5, 0)),
            pl.BlockSpec((BR, d_in), lambda i: (i, 0)),
            pl.BlockSpec((BR, 2), lambda i: (i, 0)),
            pl.BlockSpec((d_in, d_out), lambda i: (0, 0)),
            pl.BlockSpec((1, d_out), lambda i: (0, 0)),
            pl.BlockSpec((1, d_out), lambda i: (0, 0)),
        ],
        out_specs=pl.BlockSpec((BR, d_out), lambda i: (i, 0)),
        out_shape=jax.ShapeDtypeStruct((n, d_out), jnp.float32),
    )(acc, z, deg_pair, W, b.reshape(1, d_out), alpha.reshape(1, d_out))
    return out


# speedup vs baseline: 1.0205x; 1.0044x over previous
import functools

import jax
import jax.numpy as jnp
from jax import lax
from jax.experimental import pallas as pl
from jax.experimental.pallas import tpu as pltpu
from jax.experimental.pallas import tpu_sc as plsc

N_NODES = 10000
D_IN = 128
HALF = 5000
N_ACC = 10016
N_ACC_H = HALF
NUM_TILES = 32
CHUNK = 128
NCH_DEG = 80
EPT = NCH_DEG * CHUNK
CAP = EPT + CHUNK
NB = 4
NI = 8
BR = 1000


def _sc_mesh():
    return plsc.VectorSubcoreMesh(core_axis_name="c", subcore_axis_name="s")


def _part_kernel(src_hbm, dst_hbm, zeros_hbm,
                 deg_out, psrc_out, pdst_out, cnt_out,
                 srcv, dstv, lo_s, lo_d, hi_s, hi_d, ones_v, cntv, deg_sh,
                 hsem):
    cid = lax.axis_index("c")
    sid = lax.axis_index("s")
    wid = cid * 16 + sid

    @pl.when(sid == 0)
    def _():
        pltpu.sync_copy(zeros_hbm, deg_sh)

    for j in range(CHUNK // 16):
        ones_v[pl.ds(j * 16, 16)] = jnp.ones((16,), jnp.float32)
    pltpu.sync_copy(src_hbm.at[wid], srcv)
    pltpu.sync_copy(dst_hbm.at[wid], dstv)
    plsc.subcore_barrier()

    def body(c, carry):
        pltpu.async_copy(ones_v, deg_sh.at[dstv.at[c]], hsem, add=True)
        return carry

    lax.fori_loop(0, NCH_DEG, body, 0)

    def pbody(i, carry):
        lo_cnt, hi_cnt = carry
        c = i // 8
        g = (i % 8) * 16
        vd = dstv[c, pl.ds(g, 16)]
        vs = srcv[c, pl.ds(g, 16)]
        mall = jnp.ones((16,), jnp.bool_)
        vdau, vsa, _ = plsc.sort_key_val(
            plsc.bitcast(vd, jnp.uint32), vs, mask=mall)
        vda = plsc.bitcast(vdau, jnp.int32)
        vdd = lax.rev(vda, (0,))
        vsd = lax.rev(vsa, (0,))
        lo_d[pl.ds(lo_cnt, 16)] = vda
        lo_s[pl.ds(lo_cnt, 16)] = vsa
        vr = vdd - HALF
        hi_d[pl.ds(hi_cnt, 16)] = jnp.where(vr >= HALF, vr - HALF, vr)
        hi_s[pl.ds(hi_cnt, 16)] = vsd
        nlo = plsc.all_reduce_population_count(vd < HALF)[0]
        return lo_cnt + nlo, hi_cnt + (16 - nlo)

    zero = jnp.int32(0)
    lo_cnt, hi_cnt = lax.fori_loop(0, NCH_DEG * 8, pbody, (zero, zero))

    def hdrain(c, carry):
        pltpu.make_async_copy(ones_v, deg_sh.at[dstv.at[c]], hsem).wait()
        return carry

    lax.fori_loop(0, NCH_DEG, hdrain, 0)

    junk_d = jnp.arange(16, dtype=jnp.int32)
    junk_s = N_NODES + (jnp.arange(16, dtype=jnp.int32) % 8)

    def jbody(k, carry):
        lo_d[pl.ds(lo_cnt + k * 16, 16)] = junk_d
        lo_s[pl.ds(lo_cnt + k * 16, 16)] = junk_s
        hi_d[pl.ds(hi_cnt + k * 16, 16)] = junk_d
        hi_s[pl.ds(hi_cnt + k * 16, 16)] = junk_s
        return carry

    lax.fori_loop(0, CHUNK // 16, jbody, 0)

    cntv[0, pl.ds(0, 16)] = jnp.full((16,), lo_cnt, jnp.int32)
    cntv[1, pl.ds(0, 16)] = jnp.full((16,), hi_cnt, jnp.int32)
    pltpu.sync_copy(lo_s, psrc_out.at[0, wid])
    pltpu.sync_copy(lo_d, pdst_out.at[0, wid])
    pltpu.sync_copy(hi_s, psrc_out.at[1, wid])
    pltpu.sync_copy(hi_d, pdst_out.at[1, wid])
    pltpu.sync_copy(cntv, cnt_out.at[wid])
    plsc.subcore_barrier()

    @pl.when(sid == 0)
    def _():
        pltpu.sync_copy(deg_sh, deg_out.at[cid])


def _scatter_kernel(z_hbm, psrc_hbm, pdst_hbm, cnt_hbm, zeros_hbm, acc_out,
                    idxs, idxd, bufs, acc_sh, ise, gsem, ssem):
    cid = lax.axis_index("c")
    sid = lax.axis_index("s")
    p0 = 2 * sid
    p1 = 2 * sid + 1

    @pl.when(sid == 0)
    def _():
        pltpu.sync_copy(zeros_hbm, acc_sh)

    pltpu.sync_copy(cnt_hbm.at[pl.ds(sid * 64, 64)], idxs.at[0, pl.ds(0, 64)])
    plsc.subcore_barrier()

    c0 = idxs[0, pl.ds(cid * 16, 16)][0]
    c1 = idxs[0, pl.ds(32 + cid * 16, 16)][0]
    t0 = (c0 + CHUNK - 1) // CHUNK
    t1 = (c1 + CHUNK - 1) // CHUNK
    total = t0 + t1

    def step(s, carry):
        @pl.when(jnp.logical_and(s >= 8, s - 8 < total))
        def _():
            jd = lax.rem(s - 8, NB)
            pltpu.make_async_copy(
                bufs.at[jd], acc_sh.at[idxd.at[lax.rem(s - 8, NI)]],
                ssem.at[jd]).wait()

        @pl.when(s < total)
        def _():
            ji = lax.rem(s, NI)
            in0 = s < t0
            pt = jnp.where(in0, p0, p1)
            off = jnp.where(in0, s, s - t0) * CHUNK
            pltpu.async_copy(
                psrc_hbm.at[cid, pt, pl.ds(off, CHUNK)], idxs.at[ji],
                ise.at[ji])
            pltpu.async_copy(
                pdst_hbm.at[cid, pt, pl.ds(off, CHUNK)], idxd.at[ji],
                ise.at[ji])

        @pl.when(jnp.logical_and(s >= 4, s - 4 < total))
        def _():
            jg = lax.rem(s - 4, NB)
            jgi = lax.rem(s - 4, NI)
            pltpu.make_async_copy(
                psrc_hbm.at[cid, p0, pl.ds(0, CHUNK)], idxs.at[jgi],
                ise.at[jgi]).wait()
            pltpu.make_async_copy(
                pdst_hbm.at[cid, p0, pl.ds(0, CHUNK)], idxd.at[jgi],
                ise.at[jgi]).wait()
            pltpu.async_copy(
                z_hbm.at[idxs.at[jgi]], bufs.at[jg], gsem.at[jg])

        @pl.when(jnp.logical_and(s >= 7, s - 7 < total))
        def _():
            js = lax.rem(s - 7, NB)
            jsi = lax.rem(s - 7, NI)
            pltpu.make_async_copy(
                z_hbm.at[idxs.at[jsi]], bufs.at[js], gsem.at[js]).wait()
            pltpu.async_copy(
                bufs.at[js], acc_sh.at[idxd.at[jsi]], ssem.at[js],
                add=True)

        return carry

    lax.fori_loop(0, total + 8, step, 0)
    plsc.subcore_barrier()

    @pl.when(sid == 0)
    def _():
        pltpu.sync_copy(acc_sh, acc_out.at[cid])


def _scale_kernel(x_ref, d_ref, z_ref):
    deg = d_ref[:, 0:1] + d_ref[:, 1:2] + 1.0
    z_ref[...] = x_ref[...] * lax.rsqrt(deg)


def _out_kernel(a_ref, z_ref, d_ref, w_ref, b_ref, al_ref, o_ref):
    deg = d_ref[:, 0:1] + d_ref[:, 1:2] + 1.0
    s = (a_ref[0] + z_ref[...]) * lax.rsqrt(deg)
    o = jnp.dot(s, w_ref[...], preferred_element_type=jnp.float32)
    o = o + b_ref[...]
    o_ref[...] = jnp.where(o > 0, o, al_ref[...] * o)


def kernel(x, edge_index, W, b, alpha):
    n, d_in = x.shape
    e = edge_index.shape[1]
    d_out = W.shape[1]

    e_pad = NUM_TILES * EPT
    pad16 = jnp.arange(e_pad - e, dtype=jnp.int32) % 16
    src_p = jnp.concatenate([edge_index[0], n + pad16 % 8]).reshape(
        NUM_TILES, NCH_DEG, CHUNK)
    dst_p = jnp.concatenate([edge_index[1], n + pad16]).reshape(
        NUM_TILES, NCH_DEG, CHUNK)

    zeros_deg = jnp.zeros((N_ACC,), jnp.float32)
    zeros_acc = jnp.zeros((N_ACC_H, D_IN), jnp.float32)

    part_fn = pl.kernel(
        _part_kernel,
        out_type=(
            jax.ShapeDtypeStruct((2, N_ACC), jnp.float32),
            jax.ShapeDtypeStruct((2, NUM_TILES, CAP), jnp.int32),
            jax.ShapeDtypeStruct((2, NUM_TILES, CAP), jnp.int32),
            jax.ShapeDtypeStruct((NUM_TILES, 2, 16), jnp.int32),
        ),
        mesh=_sc_mesh(),
        compiler_params=pltpu.CompilerParams(needs_layout_passes=False),
        scratch_types=[
            pltpu.VMEM((NCH_DEG, CHUNK), jnp.int32),
            pltpu.VMEM((NCH_DEG, CHUNK), jnp.int32),
            pltpu.VMEM((CAP,), jnp.int32),
            pltpu.VMEM((CAP,), jnp.int32),
            pltpu.VMEM((CAP,), jnp.int32),
            pltpu.VMEM((CAP,), jnp.int32),
            pltpu.VMEM((CHUNK,), jnp.float32),
            pltpu.VMEM((2, 16), jnp.int32),
            pltpu.VMEM_SHARED((N_ACC,), jnp.float32),
            pltpu.SemaphoreType.DMA,
        ],
    )
    deg2, psrc, pdst, cnt = part_fn(src_p, dst_p, zeros_deg)

    deg_pair = deg2[:, :n].T

    grid = n // BR
    z = pl.pallas_call(
        _scale_kernel,
        grid=(grid,),
        in_specs=[
            pl.BlockSpec((BR, d_in), lambda i: (i, 0)),
            pl.BlockSpec((BR, 2), lambda i: (i, 0)),
        ],
        out_specs=pl.BlockSpec((BR, d_in), lambda i: (i, 0)),
        out_shape=jax.ShapeDtypeStruct((n, d_in), jnp.float32),
    )(x, deg_pair)

    scatter_fn = pl.kernel(
        _scatter_kernel,
        out_type=jax.ShapeDtypeStruct((2, N_ACC_H, D_IN), jnp.float32),
        mesh=_sc_mesh(),
        scratch_types=[
            pltpu.VMEM((NI, CHUNK), jnp.int32),
            pltpu.VMEM((NI, CHUNK), jnp.int32),
            pltpu.VMEM((NB, CHUNK, D_IN), jnp.float32),
            pltpu.VMEM_SHARED((N_ACC_H, D_IN), jnp.float32),
            pltpu.SemaphoreType.DMA((NI,)),
            pltpu.SemaphoreType.DMA((NB,)),
            pltpu.SemaphoreType.DMA((NB,)),
        ],
    )
    z_p = jnp.pad(z, ((0, 8), (0, 0)))
    acc = scatter_fn(z_p, psrc, pdst, cnt.reshape(-1), zeros_acc)

    out = pl.pallas_call(
        _out_kernel,
        grid=(grid,),
        in_specs=[
            pl.BlockSpec((1, BR, d_in), lambda i: (i // 5, i % 5, 0)),
            pl.BlockSpec((BR, d_in), lambda i: (i, 0)),
            pl.BlockSpec((BR, 2), lambda i: (i, 0)),
            pl.BlockSpec((d_in, d_out), lambda i: (0, 0)),
            pl.BlockSpec((1, d_out), lambda i: (0, 0)),
            pl.BlockSpec((1, d_out), lambda i: (0, 0)),
        ],
        out_specs=pl.BlockSpec((BR, d_out), lambda i: (i, 0)),
        out_shape=jax.ShapeDtypeStruct((n, d_out), jnp.float32),
    )(acc, z, deg_pair, W, b.reshape(1, d_out), alpha.reshape(1, d_out))
    return out


# speedup vs baseline: 1.0253x; 1.0047x over previous
import functools

import jax
import jax.numpy as jnp
from jax import lax
from jax.experimental import pallas as pl
from jax.experimental.pallas import tpu as pltpu
from jax.experimental.pallas import tpu_sc as plsc

N_NODES = 10000
D_IN = 128
HALF = 5000
N_ACC = 10016
N_ACC_H = 5016
NUM_TILES = 32
CHUNK = 128
NCH_DEG = 80
NCH_SC = 157
NB = 3
BR = 1000


def _sc_mesh():
    return plsc.VectorSubcoreMesh(core_axis_name="c", subcore_axis_name="s")


def _deg_kernel(dst_hbm, zeros_hbm, deg_out, idx_v, ones_v, deg_sh, sem):
    cid = lax.axis_index("c")
    sid = lax.axis_index("s")
    wid = cid * 16 + sid

    @pl.when(sid == 0)
    def _():
        pltpu.sync_copy(zeros_hbm, deg_sh)

    for j in range(CHUNK // 16):
        ones_v[pl.ds(j * 16, 16)] = jnp.ones((16,), jnp.float32)
    pltpu.sync_copy(dst_hbm.at[wid], idx_v)
    plsc.subcore_barrier()

    def body(c, carry):
        pltpu.sync_copy(ones_v, deg_sh.at[idx_v.at[c]], add=True)
        return carry

    lax.fori_loop(0, NCH_DEG, body, 0)
    plsc.subcore_barrier()

    @pl.when(sid == 0)
    def _():
        pltpu.sync_copy(deg_sh, deg_out.at[cid])


LA = 2


def _scatter_kernel(z_hbm, src_hbm, dst_hbm, zeros_hbm, acc_out,
                    srcv, dstv, bufs, acc_sh, gsem, ssem):
    cid = lax.axis_index("c")
    sid = lax.axis_index("s")
    wid = cid * 16 + sid

    @pl.when(sid == 0)
    def _():
        pltpu.sync_copy(zeros_hbm, acc_sh)

    pltpu.sync_copy(src_hbm.at[wid], srcv)
    pltpu.sync_copy(dst_hbm.at[wid], dstv)
    plsc.subcore_barrier()

    def step(s, carry):
        j = lax.rem(s, NB)

        @pl.when(jnp.logical_and(s >= NB, s - NB < NCH_SC))
        def _():
            pltpu.make_async_copy(
                bufs.at[j], acc_sh.at[dstv.at[s - NB]], ssem.at[j]).wait()

        @pl.when(s < NCH_SC)
        def _():
            pltpu.async_copy(z_hbm.at[srcv.at[s]], bufs.at[j], gsem.at[j])

        @pl.when(jnp.logical_and(s >= LA, s - LA < NCH_SC))
        def _():
            c = s - LA
            jc = lax.rem(c, NB)
            pltpu.make_async_copy(
                z_hbm.at[srcv.at[c]], bufs.at[jc], gsem.at[jc]).wait()
            pltpu.async_copy(
                bufs.at[jc], acc_sh.at[dstv.at[c]], ssem.at[jc], add=True)

        return carry

    lax.fori_loop(0, NCH_SC + NB, step, 0)
    plsc.subcore_barrier()

    @pl.when(sid == 0)
    def _():
        pltpu.sync_copy(acc_sh, acc_out.at[cid])


def _scale_kernel(x_ref, d_ref, z_ref):
    deg = d_ref[:, 0:1] + d_ref[:, 1:2] + 1.0
    z_ref[...] = x_ref[...] * lax.rsqrt(deg)


def _out_kernel(a_ref, z_ref, d_ref, w_ref, b_ref, al_ref, o_ref):
    deg = d_ref[:, 0:1] + d_ref[:, 1:2] + 1.0
    s = (a_ref[0] + z_ref[...]) * lax.rsqrt(deg)
    o = jnp.dot(s, w_ref[...], preferred_element_type=jnp.float32)
    o = o + b_ref[...]
    o_ref[...] = jnp.where(o > 0, o, al_ref[...] * o)


def kernel(x, edge_index, W, b, alpha):
    n, d_in = x.shape
    e = edge_index.shape[1]
    d_out = W.shape[1]

    e_pad_deg = NUM_TILES * NCH_DEG * CHUNK
    pad16 = jnp.arange(e_pad_deg - e, dtype=jnp.int32) % 16
    dst_deg = jnp.concatenate([edge_index[1], n + pad16]).reshape(
        NUM_TILES, NCH_DEG, CHUNK)

    e_pad_sc = 16 * NCH_SC * CHUNK
    padsc = jnp.arange(e_pad_sc - e, dtype=jnp.int32) % 16
    src_h = jnp.concatenate([edge_index[0], padsc]).reshape(
        1, 16, NCH_SC, CHUNK)
    src2 = jnp.concatenate([src_h, src_h], axis=0).reshape(
        NUM_TILES, NCH_SC, CHUNK)
    dstp = jnp.concatenate([edge_index[1], n + padsc])
    junk = HALF + (jnp.arange(dstp.shape[0], dtype=jnp.int32) % 16)
    dst_lo = jnp.where(dstp < HALF, dstp, junk)
    dst_hi = jnp.where(dstp >= HALF, dstp - HALF, junk)
    dst2 = jnp.concatenate(
        [dst_lo.reshape(1, 16, NCH_SC, CHUNK),
         dst_hi.reshape(1, 16, NCH_SC, CHUNK)], axis=0
    ).reshape(NUM_TILES, NCH_SC, CHUNK)

    zeros_deg = jnp.zeros((N_ACC,), jnp.float32)
    zeros_acc = jnp.zeros((N_ACC_H, D_IN), jnp.float32)

    deg_fn = pl.kernel(
        _deg_kernel,
        out_type=jax.ShapeDtypeStruct((2, N_ACC), jnp.float32),
        mesh=_sc_mesh(),
        scratch_types=[
            pltpu.VMEM((NCH_DEG, CHUNK), jnp.int32),
            pltpu.VMEM((CHUNK,), jnp.float32),
            pltpu.VMEM_SHARED((N_ACC,), jnp.float32),
            pltpu.SemaphoreType.DMA,
        ],
    )
    deg2 = deg_fn(dst_deg, zeros_deg)

    deg_pair = deg2[:, :n].T

    grid = n // BR
    z = pl.pallas_call(
        _scale_kernel,
        grid=(grid,),
        in_specs=[
            pl.BlockSpec((BR, d_in), lambda i: (i, 0)),
            pl.BlockSpec((BR, 2), lambda i: (i, 0)),
        ],
        out_specs=pl.BlockSpec((BR, d_in), lambda i: (i, 0)),
        out_shape=jax.ShapeDtypeStruct((n, d_in), jnp.float32),
    )(x, deg_pair)

    scatter_fn = pl.kernel(
        _scatter_kernel,
        out_type=jax.ShapeDtypeStruct((2, N_ACC_H, D_IN), jnp.float32),
        mesh=_sc_mesh(),
        scratch_types=[
            pltpu.VMEM((NCH_SC, CHUNK), jnp.int32),
            pltpu.VMEM((NCH_SC, CHUNK), jnp.int32),
            pltpu.VMEM((NB, CHUNK, D_IN), jnp.float32),
            pltpu.VMEM_SHARED((N_ACC_H, D_IN), jnp.float32),
            pltpu.SemaphoreType.DMA((NB,)),
            pltpu.SemaphoreType.DMA((NB,)),
        ],
    )
    acc = scatter_fn(z, src2, dst2, zeros_acc)

    out = pl.pallas_call(
        _out_kernel,
        grid=(grid,),
        in_specs=[
            pl.BlockSpec((1, BR, d_in), lambda i: (i // 5, i % 5, 0)),
            pl.BlockSpec((BR, d_in), lambda i: (i, 0)),
            pl.BlockSpec((BR, 2), lambda i: (i, 0)),
            pl.BlockSpec((d_in, d_out), lambda i: (0, 0)),
            pl.BlockSpec((1, d_out), lambda i: (0, 0)),
            pl.BlockSpec((1, d_out), lambda i: (0, 0)),
        ],
        out_specs=pl.BlockSpec((BR, d_out), lambda i: (i, 0)),
        out_shape=jax.ShapeDtypeStruct((n, d_out), jnp.float32),
    )(acc, z, deg_pair, W, b.reshape(1, d_out), alpha.reshape(1, d_out))
    return out
